# Initial kernel scaffold; baseline (speedup 1.0000x reference)
import jax
import jax.numpy as jnp
from jax.experimental import pallas as pl


def kernel(img_global_feat, image_patch_feats, params):
    raise NotImplementedError("write your pallas kernel here")


# speedup vs baseline: 1.9069x; 1.9069x over previous
import functools
import math

import jax
import jax.numpy as jnp
from jax.experimental import pallas as pl

H = 8
TOPK = 8
EPS = 0.01


def _ln(x, s, b):
    m = jnp.mean(x, axis=-1, keepdims=True)
    d = x - m
    v = jnp.mean(d * d, axis=-1, keepdims=True)
    return d / jnp.sqrt(v + 1e-5) * s + b


def _lnmm_kernel(x_ref, s_ref, b_ref, w_ref, o_ref):
    h = _ln(x_ref[...], s_ref[...], b_ref[...])
    o_ref[...] = jnp.dot(h, w_ref[...], preferred_element_type=jnp.float32)


def _lnmm(x, s, b, w, bm=256):
    M, D = x.shape
    N = w.shape[1]
    return pl.pallas_call(
        _lnmm_kernel,
        grid=(M // bm,),
        in_specs=[
            pl.BlockSpec((bm, D), lambda i: (i, 0)),
            pl.BlockSpec((1, D), lambda i: (0, 0)),
            pl.BlockSpec((1, D), lambda i: (0, 0)),
            pl.BlockSpec((D, N), lambda i: (0, 0)),
        ],
        out_specs=pl.BlockSpec((bm, N), lambda i: (i, 0)),
        out_shape=jax.ShapeDtypeStruct((M, N), jnp.float32),
    )(x, s.reshape(1, D), b.reshape(1, D), w)


def _attn_kernel(nreal, scale, q_ref, k_ref, v_ref, o_ref):
    q = q_ref[0]
    k = k_ref[0]
    v = v_ref[0]
    s = jax.lax.dot_general(q, k, (((1,), (1,)), ((), ())),
                            preferred_element_type=jnp.float32) * scale
    mask = jax.lax.broadcasted_iota(jnp.int32, s.shape, 1) < nreal
    s = jnp.where(mask, s, -1e30)
    s = s - jnp.max(s, axis=-1, keepdims=True)
    e = jnp.exp(s)
    p = e / jnp.sum(e, axis=-1, keepdims=True)
    o_ref[0] = jnp.dot(p, v, preferred_element_type=jnp.float32)


def _attention(qa, ka, va, oq, ok, ov, nreal, hd):
    B, Nq, _ = qa.shape
    Nk = ka.shape[1]
    kern = functools.partial(_attn_kernel, nreal, 1.0 / math.sqrt(hd))
    return pl.pallas_call(
        kern,
        grid=(B, H),
        in_specs=[
            pl.BlockSpec((1, Nq, hd), lambda b, h: (b, 0, oq + h)),
            pl.BlockSpec((1, Nk, hd), lambda b, h: (b, 0, ok + h)),
            pl.BlockSpec((1, Nk, hd), lambda b, h: (b, 0, ov + h)),
        ],
        out_specs=pl.BlockSpec((1, Nq, hd), lambda b, h: (b, 0, h)),
        out_shape=jax.ShapeDtypeStruct((B, Nq, H * hd), jnp.float32),
    )(qa, ka, va)


def _resmm_kernel(x_ref, a_ref, w_ref, o_ref):
    o_ref[...] = x_ref[...] + jnp.dot(a_ref[...], w_ref[...],
                                      preferred_element_type=jnp.float32)


def _resmm(x, a, w, bm=256):
    M, D = x.shape
    N = w.shape[1]
    return pl.pallas_call(
        _resmm_kernel,
        grid=(M // bm,),
        in_specs=[
            pl.BlockSpec((bm, D), lambda i: (i, 0)),
            pl.BlockSpec((bm, D), lambda i: (i, 0)),
            pl.BlockSpec((D, N), lambda i: (0, 0)),
        ],
        out_specs=pl.BlockSpec((bm, N), lambda i: (i, 0)),
        out_shape=jax.ShapeDtypeStruct((M, N), jnp.float32),
    )(x, a, w)


def _mlp_kernel(x_ref, s_ref, b_ref, w1_ref, b1_ref, w2_ref, b2_ref, o_ref):
    x = x_ref[...]
    h = _ln(x, s_ref[...], b_ref[...])
    u = jnp.dot(h, w1_ref[...], preferred_element_type=jnp.float32) + b1_ref[...]
    u = jax.nn.gelu(u)
    o_ref[...] = x + jnp.dot(u, w2_ref[...],
                             preferred_element_type=jnp.float32) + b2_ref[...]


def _mlp(x, s, b, w1, b1, w2, b2, bm=256):
    M, D = x.shape
    F = w1.shape[1]
    return pl.pallas_call(
        _mlp_kernel,
        grid=(M // bm,),
        in_specs=[
            pl.BlockSpec((bm, D), lambda i: (i, 0)),
            pl.BlockSpec((1, D), lambda i: (0, 0)),
            pl.BlockSpec((1, D), lambda i: (0, 0)),
            pl.BlockSpec((D, F), lambda i: (0, 0)),
            pl.BlockSpec((1, F), lambda i: (0, 0)),
            pl.BlockSpec((F, D), lambda i: (0, 0)),
            pl.BlockSpec((1, D), lambda i: (0, 0)),
        ],
        out_specs=pl.BlockSpec((bm, D), lambda i: (i, 0)),
        out_shape=jax.ShapeDtypeStruct((M, D), jnp.float32),
    )(x, s.reshape(1, D), b.reshape(1, D), w1, b1.reshape(1, F), w2,
      b2.reshape(1, D))


def _head_kernel(B, K, x_ref, w_ref, b_ref, g_ref, sel_ref, nr_ref):
    lat = jax.nn.sigmoid(
        jnp.dot(x_ref[...], w_ref[...], preferred_element_type=jnp.float32)
        + b_ref[...])
    lat3 = lat.reshape(B, K, lat.shape[-1])
    lb = lat3.astype(jnp.bfloat16).astype(jnp.float32)
    gb = g_ref[...].astype(jnp.bfloat16).astype(jnp.float32)
    aw = jnp.sum(lb * gb[:, None, :], axis=-1)
    aw = aw - jnp.max(aw, axis=1, keepdims=True)
    e = jnp.exp(aw)
    aw = e / jnp.sum(e, axis=1, keepdims=True)
    count = jnp.sum((aw > EPS).astype(jnp.int32), axis=1, keepdims=True)
    num_r = jnp.clip(count, 1, TOPK)
    ai = aw[:, :, None]
    aj = aw[:, None, :]
    ii = jax.lax.broadcasted_iota(jnp.int32, (B, K, K), 1)
    jj = jax.lax.broadcasted_iota(jnp.int32, (B, K, K), 2)
    cmp = (aj > ai) | ((aj == ai) & (jj < ii))
    rank = jnp.sum(cmp.astype(jnp.int32), axis=2)
    is_sel = rank < TOPK
    is_topr = rank < num_r
    prefix = jnp.sum((is_topr[:, None, :] & (jj < ii)).astype(jnp.int32),
                     axis=2)
    out_pos = jnp.where(is_topr, prefix, rank)
    bidx = jax.lax.broadcasted_iota(jnp.int32, (B, K), 0)
    g_out = jnp.where(is_sel, bidx * TOPK + out_pos, -1).reshape(1, B * K)
    rr = jax.lax.broadcasted_iota(jnp.int32, (B * TOPK, B * K), 0)
    onehot = (g_out == rr).astype(jnp.float32)
    sel = jnp.dot(onehot, lat, preferred_element_type=jnp.float32,
                  precision=jax.lax.Precision.HIGHEST)
    nrm = jnp.sqrt(jnp.sum(sel * sel, axis=-1, keepdims=True))
    sel_ref[...] = sel / jnp.maximum(nrm, 1e-12)
    nr_ref[...] = jnp.broadcast_to(num_r, (B, 128))


def _head(x, fc_w, fc_b, g, B, K):
    T = fc_w.shape[1]
    kern = functools.partial(_head_kernel, B, K)
    return pl.pallas_call(
        kern,
        out_shape=(jax.ShapeDtypeStruct((B * TOPK, T), jnp.float32),
                   jax.ShapeDtypeStruct((B, 128), jnp.int32)),
    )(x, fc_w, fc_b.reshape(1, T), g)


def kernel(img_global_feat, image_patch_feats, params):
    Bs, P, D = image_patch_feats.shape
    K = params["templates"].shape[1]
    hd = D // H
    n_real = K + P
    Np = ((n_real + 127) // 128) * 128
    M = Bs * Np

    tmpl = jnp.broadcast_to(params["templates"], (Bs, K, D))
    x = jnp.concatenate(
        [tmpl, image_patch_feats,
         jnp.zeros((Bs, Np - n_real, D), jnp.float32)], axis=1)
    xf = x.reshape(M, D)

    p = params["layers"][0]
    wqkv = jnp.concatenate([p["Wq"], p["Wk"], p["Wv"]], axis=1)
    qkv = _lnmm(xf, p["ln1_s"], p["ln1_b"], wqkv).reshape(Bs, Np, 3 * D)
    o = _attention(qkv, qkv, qkv, 0, H, 2 * H, n_real, hd)
    x1 = _resmm(xf, o.reshape(M, D), p["Wo"])
    x2 = _mlp(x1, p["ln2_s"], p["ln2_b"], p["W1"], p["b1"], p["W2"], p["b2"])

    p = params["layers"][1]
    wkv = jnp.concatenate([p["Wk"], p["Wv"]], axis=1)
    kv = _lnmm(x2, p["ln1_s"], p["ln1_b"], wkv).reshape(Bs, Np, 2 * D)
    xs = x2.reshape(Bs, Np, D)[:, :K, :].reshape(Bs * K, D)
    q = _lnmm(xs, p["ln1_s"], p["ln1_b"], p["Wq"]).reshape(Bs, K, D)
    o2 = _attention(q, kv, kv, 0, 0, H, n_real, hd)
    x3 = _resmm(xs, o2.reshape(Bs * K, D), p["Wo"])
    x4 = _mlp(x3, p["ln2_s"], p["ln2_b"], p["W1"], p["b1"], p["W2"], p["b2"])

    sel, nr = _head(x4, params["fc_W"], params["fc_b"], img_global_feat, Bs, K)
    T = params["fc_W"].shape[1]
    return sel.reshape(Bs, TOPK, T), nr[:, 0]


# speedup vs baseline: 2.6214x; 1.3747x over previous
import functools
import math

import jax
import jax.numpy as jnp
from jax.experimental import pallas as pl
from jax.experimental.pallas import tpu as pltpu

H = 8
TOPK = 8
EPS = 0.01


def _ln(x, s, b):
    m = jnp.mean(x, axis=-1, keepdims=True)
    d = x - m
    v = jnp.mean(d * d, axis=-1, keepdims=True)
    return d / jnp.sqrt(v + 1e-5) * s + b


def _sm_rows(s):
    s = s - jnp.max(s, axis=-1, keepdims=True)
    e = jnp.exp(s)
    return e / jnp.sum(e, axis=-1, keepdims=True)


def _heads_attn(q, k, v, hd, nreal, o_scr):
    nq = q.shape[0]
    for i in range(H):
        qh = q[:, i * hd:(i + 1) * hd]
        kh = k[:, i * hd:(i + 1) * hd]
        vh = v[:, i * hd:(i + 1) * hd]
        s = jax.lax.dot_general(qh, kh, (((1,), (1,)), ((), ())),
                                preferred_element_type=jnp.float32)
        s = s / jnp.sqrt(jnp.float32(hd))
        mask = jax.lax.broadcasted_iota(jnp.int32, s.shape, 1) < nreal
        p = _sm_rows(jnp.where(mask, s, -1e30))
        o_scr[:, i * hd:(i + 1) * hd] = jnp.dot(
            p, vh, preferred_element_type=jnp.float32)
    return o_scr[...]


def _ab1_kernel(nreal, npad, hd, tmpl_ref, patch_ref, s_ref, b_ref, wq_ref,
                wk_ref, wv_ref, wo_ref, o_ref, x_scr, o_scr):
    K = tmpl_ref.shape[0]
    D = tmpl_ref.shape[1]
    x_scr[0:K, :] = tmpl_ref[...]
    x_scr[K:nreal, :] = patch_ref[0]
    x_scr[nreal:npad, :] = jnp.zeros((npad - nreal, D), jnp.float32)
    x = x_scr[...]
    h = _ln(x, s_ref[...], b_ref[...])
    q = jnp.dot(h, wq_ref[...], preferred_element_type=jnp.float32)
    k = jnp.dot(h, wk_ref[...], preferred_element_type=jnp.float32)
    v = jnp.dot(h, wv_ref[...], preferred_element_type=jnp.float32)
    o = _heads_attn(q, k, v, hd, nreal, o_scr)
    o_ref[0] = x + jnp.dot(o, wo_ref[...], preferred_element_type=jnp.float32)


def _attn_block1(tmpl, patches, s, b, wq, wk, wv, wo, npad):
    B, P, D = patches.shape
    nreal = tmpl.shape[0] + P
    hd = D // H
    kern = functools.partial(_ab1_kernel, nreal, npad, hd)
    return pl.pallas_call(
        kern,
        grid=(B,),
        in_specs=[
            pl.BlockSpec(tmpl.shape, lambda i: (0, 0)),
            pl.BlockSpec((1, P, D), lambda i: (i, 0, 0)),
            pl.BlockSpec((1, D), lambda i: (0, 0)),
            pl.BlockSpec((1, D), lambda i: (0, 0)),
            pl.BlockSpec((D, D), lambda i: (0, 0)),
            pl.BlockSpec((D, D), lambda i: (0, 0)),
            pl.BlockSpec((D, D), lambda i: (0, 0)),
            pl.BlockSpec((D, D), lambda i: (0, 0)),
        ],
        out_specs=pl.BlockSpec((1, npad, D), lambda i: (i, 0, 0)),
        out_shape=jax.ShapeDtypeStruct((B, npad, D), jnp.float32),
        scratch_shapes=[pltpu.VMEM((npad, D), jnp.float32),
                        pltpu.VMEM((npad, D), jnp.float32)],
    )(tmpl, patches, s.reshape(1, D), b.reshape(1, D), wq, wk, wv, wo)


def _ab2_kernel(nreal, K, hd, x_ref, s_ref, b_ref, wq_ref, wk_ref, wv_ref,
                wo_ref, o_ref, o_scr):
    x = x_ref[0]
    h = _ln(x, s_ref[...], b_ref[...])
    k = jnp.dot(h, wk_ref[...], preferred_element_type=jnp.float32)
    v = jnp.dot(h, wv_ref[...], preferred_element_type=jnp.float32)
    q = jnp.dot(h[0:K, :], wq_ref[...], preferred_element_type=jnp.float32)
    o = _heads_attn(q, k, v, hd, nreal, o_scr)
    o_ref[0] = x[0:K, :] + jnp.dot(o, wo_ref[...],
                                   preferred_element_type=jnp.float32)


def _attn_block2(x, s, b, wq, wk, wv, wo, nreal, K):
    B, npad, D = x.shape
    hd = D // H
    kern = functools.partial(_ab2_kernel, nreal, K, hd)
    return pl.pallas_call(
        kern,
        grid=(B,),
        in_specs=[
            pl.BlockSpec((1, npad, D), lambda i: (i, 0, 0)),
            pl.BlockSpec((1, D), lambda i: (0, 0)),
            pl.BlockSpec((1, D), lambda i: (0, 0)),
            pl.BlockSpec((D, D), lambda i: (0, 0)),
            pl.BlockSpec((D, D), lambda i: (0, 0)),
            pl.BlockSpec((D, D), lambda i: (0, 0)),
            pl.BlockSpec((D, D), lambda i: (0, 0)),
        ],
        out_specs=pl.BlockSpec((1, K, D), lambda i: (i, 0, 0)),
        out_shape=jax.ShapeDtypeStruct((B, K, D), jnp.float32),
        scratch_shapes=[pltpu.VMEM((K, D), jnp.float32)],
    )(x, s.reshape(1, D), b.reshape(1, D), wq, wk, wv, wo)


def _mlp_kernel(x_ref, s_ref, b_ref, w1_ref, b1_ref, w2_ref, b2_ref, o_ref):
    x = x_ref[...]
    h = _ln(x, s_ref[...], b_ref[...])
    u = jnp.dot(h, w1_ref[...], preferred_element_type=jnp.float32) + b1_ref[...]
    u = jax.nn.gelu(u)
    o_ref[...] = x + jnp.dot(u, w2_ref[...],
                             preferred_element_type=jnp.float32) + b2_ref[...]


def _mlp(x, s, b, w1, b1, w2, b2, bm=256):
    M, D = x.shape
    F = w1.shape[1]
    return pl.pallas_call(
        _mlp_kernel,
        grid=(M // bm,),
        in_specs=[
            pl.BlockSpec((bm, D), lambda i: (i, 0)),
            pl.BlockSpec((1, D), lambda i: (0, 0)),
            pl.BlockSpec((1, D), lambda i: (0, 0)),
            pl.BlockSpec((D, F), lambda i: (0, 0)),
            pl.BlockSpec((1, F), lambda i: (0, 0)),
            pl.BlockSpec((F, D), lambda i: (0, 0)),
            pl.BlockSpec((1, D), lambda i: (0, 0)),
        ],
        out_specs=pl.BlockSpec((bm, D), lambda i: (i, 0)),
        out_shape=jax.ShapeDtypeStruct((M, D), jnp.float32),
    )(x, s.reshape(1, D), b.reshape(1, D), w1, b1.reshape(1, F), w2,
      b2.reshape(1, D))


def _head_kernel(B, K, x_ref, w_ref, b_ref, g_ref, sel_ref, nr_ref):
    lat = jax.nn.sigmoid(
        jnp.dot(x_ref[...], w_ref[...], preferred_element_type=jnp.float32)
        + b_ref[...])
    lat3 = lat.reshape(B, K, lat.shape[-1])
    lb = lat3.astype(jnp.bfloat16).astype(jnp.float32)
    gb = g_ref[...].astype(jnp.bfloat16).astype(jnp.float32)
    aw = jnp.sum(lb * gb[:, None, :], axis=-1)
    aw = aw - jnp.max(aw, axis=1, keepdims=True)
    e = jnp.exp(aw)
    aw = e / jnp.sum(e, axis=1, keepdims=True)
    count = jnp.sum((aw > EPS).astype(jnp.int32), axis=1, keepdims=True)
    num_r = jnp.clip(count, 1, TOPK)
    ai = aw[:, :, None]
    aj = aw[:, None, :]
    ii = jax.lax.broadcasted_iota(jnp.int32, (B, K, K), 1)
    jj = jax.lax.broadcasted_iota(jnp.int32, (B, K, K), 2)
    cmp = (aj > ai) | ((aj == ai) & (jj < ii))
    rank = jnp.sum(cmp.astype(jnp.int32), axis=2)
    is_sel = rank < TOPK
    is_topr = rank < num_r
    prefix = jnp.sum((is_topr[:, None, :] & (jj < ii)).astype(jnp.int32),
                     axis=2)
    out_pos = jnp.where(is_topr, prefix, rank)
    bidx = jax.lax.broadcasted_iota(jnp.int32, (B, K), 0)
    g_out = jnp.where(is_sel, bidx * TOPK + out_pos, -1).reshape(1, B * K)
    rr = jax.lax.broadcasted_iota(jnp.int32, (B * TOPK, B * K), 0)
    onehot = (g_out == rr).astype(jnp.float32)
    sel = jnp.dot(onehot, lat, preferred_element_type=jnp.float32,
                  precision=jax.lax.Precision.HIGHEST)
    nrm = jnp.sqrt(jnp.sum(sel * sel, axis=-1, keepdims=True))
    sel_ref[...] = sel / jnp.maximum(nrm, 1e-12)
    nr_ref[...] = jnp.broadcast_to(num_r, (B, 128))


def _head(x, fc_w, fc_b, g, B, K):
    T = fc_w.shape[1]
    kern = functools.partial(_head_kernel, B, K)
    return pl.pallas_call(
        kern,
        out_shape=(jax.ShapeDtypeStruct((B * TOPK, T), jnp.float32),
                   jax.ShapeDtypeStruct((B, 128), jnp.int32)),
    )(x, fc_w, fc_b.reshape(1, T), g)


def kernel(img_global_feat, image_patch_feats, params):
    Bs, P, D = image_patch_feats.shape
    K = params["templates"].shape[1]
    n_real = K + P
    Np = ((n_real + 127) // 128) * 128
    tmpl = params["templates"].reshape(K, D)

    p = params["layers"][0]
    x1 = _attn_block1(tmpl, image_patch_feats, p["ln1_s"], p["ln1_b"],
                      p["Wq"], p["Wk"], p["Wv"], p["Wo"], Np)
    x2 = _mlp(x1.reshape(Bs * Np, D), p["ln2_s"], p["ln2_b"], p["W1"],
              p["b1"], p["W2"], p["b2"])

    p = params["layers"][1]
    x3 = _attn_block2(x2.reshape(Bs, Np, D), p["ln1_s"], p["ln1_b"], p["Wq"],
                      p["Wk"], p["Wv"], p["Wo"], n_real, K)
    x4 = _mlp(x3.reshape(Bs * K, D), p["ln2_s"], p["ln2_b"], p["W1"],
              p["b1"], p["W2"], p["b2"])

    sel, nr = _head(x4, params["fc_W"], params["fc_b"], img_global_feat, Bs, K)
    T = params["fc_W"].shape[1]
    return sel.reshape(Bs, TOPK, T), nr[:, 0]
